# baseline (device time: 11039 ns/iter reference)
import numpy as np

import jax
import jax.numpy as jnp
from jax import lax
from jax.experimental import pallas as pl
from jax.experimental.pallas import tpu as pltpu

M = 512
N = 256
CH = 64
NCH = M // CH

_TRI = np.triu(np.ones((M, M), np.float32), 1)


def kernel(x, dest):
    dest2 = dest.reshape(1, M)
    tri = jnp.asarray(_TRI)

    def body(x_ref, dest_ref, tri_ref, out_ref, sbuf_ref, rbuf_ref,
             send_sems, recv_sems):
        my_x = lax.axis_index("x")
        my_y = lax.axis_index("y")
        other = 1 - my_x

        barrier_sem = pltpu.get_barrier_semaphore()
        pl.semaphore_signal(
            barrier_sem, inc=1,
            device_id=(other, my_y), device_id_type=pl.DeviceIdType.MESH,
        )
        pl.semaphore_wait(barrier_sem, 1)

        dl = dest_ref[...]
        keep = dl == my_x
        maskf = keep.astype(jnp.float32)
        ck = jnp.dot(maskf, tri_ref[...], preferred_element_type=jnp.float32)
        ck = ck.astype(jnp.int32)
        il = lax.broadcasted_iota(jnp.int32, (1, M), 1)
        n_keep = jnp.sum(keep.astype(jnp.int32))
        n_mov = M - n_keep
        off = my_x * n_mov
        target = jnp.where(keep, ck, M + il - ck) + off

        xb = x_ref[...].astype(jnp.bfloat16)
        jrow = lax.broadcasted_iota(jnp.int32, (M, M), 0)

        ps = (jrow + M == target).astype(jnp.bfloat16)
        sbuf_ref[...] = jnp.dot(
            ps, xb, preferred_element_type=jnp.float32
        )

        nc = (n_mov + CH - 1) // CH

        for k in range(NCH):
            o_send = jnp.where(my_x == 0, CH * k, M - CH * (k + 1))

            @pl.when(k < nc)
            def _():
                pltpu.make_async_remote_copy(
                    src_ref=sbuf_ref.at[pl.ds(o_send, CH), :],
                    dst_ref=rbuf_ref.at[pl.ds(o_send, CH), :],
                    send_sem=send_sems.at[k],
                    recv_sem=recv_sems.at[k],
                    device_id=(other, my_y),
                    device_id_type=pl.DeviceIdType.MESH,
                ).start()

        pk = (jrow == target).astype(jnp.bfloat16)
        keep_rows = jnp.dot(pk, xb, preferred_element_type=jnp.float32)

        for k in range(NCH):
            o_send = jnp.where(my_x == 0, CH * k, M - CH * (k + 1))

            @pl.when(k < nc)
            def _():
                pltpu.make_async_remote_copy(
                    src_ref=sbuf_ref.at[pl.ds(o_send, CH), :],
                    dst_ref=rbuf_ref.at[pl.ds(o_send, CH), :],
                    send_sem=send_sems.at[k],
                    recv_sem=recv_sems.at[k],
                    device_id=(other, my_y),
                    device_id_type=pl.DeviceIdType.MESH,
                ).wait_send()

        for k in range(NCH):
            o_recv = jnp.where(my_x == 1, CH * k, M - CH * (k + 1))

            @pl.when(k < nc)
            def _():
                pltpu.make_async_remote_copy(
                    src_ref=sbuf_ref.at[pl.ds(o_recv, CH), :],
                    dst_ref=rbuf_ref.at[pl.ds(o_recv, CH), :],
                    send_sem=send_sems.at[k],
                    recv_sem=recv_sems.at[k],
                    device_id=(other, my_y),
                    device_id_type=pl.DeviceIdType.MESH,
                ).wait_recv()

        jo = lax.broadcasted_iota(jnp.int32, (M, 1), 0)
        in_keep = (jo >= off) & (jo < off + n_keep)
        out_ref[...] = jnp.where(in_keep, keep_rows, rbuf_ref[...])

    return pl.pallas_call(
        body,
        out_shape=jax.ShapeDtypeStruct((M, N), jnp.float32),
        in_specs=[
            pl.BlockSpec(memory_space=pltpu.VMEM),
            pl.BlockSpec(memory_space=pltpu.VMEM),
            pl.BlockSpec(memory_space=pltpu.VMEM),
        ],
        out_specs=pl.BlockSpec(memory_space=pltpu.VMEM),
        scratch_shapes=[
            pltpu.VMEM((M, N), jnp.float32),
            pltpu.VMEM((M, N), jnp.float32),
            pltpu.SemaphoreType.DMA((NCH,)),
            pltpu.SemaphoreType.DMA((NCH,)),
        ],
        compiler_params=pltpu.CompilerParams(collective_id=0),
    )(x, dest2, tri)


# device time: 10188 ns/iter; 1.0835x vs baseline; 1.0835x over previous
import jax
import jax.numpy as jnp
from jax import lax
from jax.experimental import pallas as pl
from jax.experimental.pallas import tpu as pltpu

M = 512
N = 256
CH = 64
NCH = M // CH


def kernel(x, dest):
    dest2 = dest.reshape(1, M)

    def body(x_ref, dest_ref, out_ref, sbuf_ref, rbuf_ref,
             send_sems, recv_sems):
        my_x = lax.axis_index("x")
        my_y = lax.axis_index("y")
        other = 1 - my_x

        barrier_sem = pltpu.get_barrier_semaphore()
        pl.semaphore_signal(
            barrier_sem, inc=1,
            device_id=(other, my_y), device_id_type=pl.DeviceIdType.MESH,
        )

        dl = dest_ref[...]
        keep = dl == my_x
        mi = keep.astype(jnp.int32)
        c = mi
        s = 1
        while s < M:
            c = c + jnp.concatenate(
                [jnp.zeros((1, s), jnp.int32), c[:, : M - s]], axis=1
            )
            s *= 2
        ck = c - mi
        il = lax.broadcasted_iota(jnp.int32, (1, M), 1)
        n_keep = jnp.sum(mi)
        n_mov = M - n_keep
        nc = (n_mov + CH - 1) // CH
        off_keep = my_x * n_mov
        off_send = my_x * (nc * CH - n_mov)
        dstbase = my_x * (M - nc * CH)

        target = jnp.where(keep, ck + off_keep, M + il - ck + off_send)
        target16 = target.astype(jnp.int16)
        xb = x_ref[...].astype(jnp.bfloat16)
        jrow = lax.broadcasted_iota(jnp.int16, (M, M), 0)

        def start_chunk(k):
            @pl.when(k < nc)
            def _():
                pltpu.make_async_remote_copy(
                    src_ref=sbuf_ref.at[pl.ds(CH * k, CH), :],
                    dst_ref=rbuf_ref.at[pl.ds(dstbase + CH * k, CH), :],
                    send_sem=send_sems.at[k],
                    recv_sem=recv_sems.at[k],
                    device_id=(other, my_y),
                    device_id_type=pl.DeviceIdType.MESH,
                ).start()

        HEAD = 2 * CH
        jrow_h = lax.broadcasted_iota(jnp.int16, (HEAD, M), 0)
        ps_h = (jrow_h + jnp.int16(M) == target16).astype(jnp.bfloat16)
        sbuf_ref[: HEAD, :] = jnp.dot(
            ps_h, xb, preferred_element_type=jnp.float32
        )

        pl.semaphore_wait(barrier_sem, 1)

        for k in range(2):
            start_chunk(k)

        jrow_t = lax.broadcasted_iota(jnp.int16, (M - HEAD, M), 0)
        ps_t = (jrow_t + jnp.int16(M + HEAD) == target16).astype(jnp.bfloat16)
        sbuf_ref[HEAD:, :] = jnp.dot(
            ps_t, xb, preferred_element_type=jnp.float32
        )
        for k in range(2, NCH):
            start_chunk(k)

        pk = (jrow == target16).astype(jnp.bfloat16)
        keep_rows = jnp.dot(pk, xb, preferred_element_type=jnp.float32)

        for k in range(NCH):
            @pl.when(k < nc)
            def _():
                pltpu.make_async_remote_copy(
                    src_ref=sbuf_ref.at[pl.ds(CH * k, CH), :],
                    dst_ref=rbuf_ref.at[pl.ds(dstbase + CH * k, CH), :],
                    send_sem=send_sems.at[k],
                    recv_sem=recv_sems.at[k],
                    device_id=(other, my_y),
                    device_id_type=pl.DeviceIdType.MESH,
                ).wait_send()

        for k in range(NCH):
            @pl.when(k < nc)
            def _():
                pltpu.make_async_remote_copy(
                    src_ref=sbuf_ref.at[pl.ds(CH * k, CH), :],
                    dst_ref=rbuf_ref.at[pl.ds(CH * k, CH), :],
                    send_sem=send_sems.at[k],
                    recv_sem=recv_sems.at[k],
                    device_id=(other, my_y),
                    device_id_type=pl.DeviceIdType.MESH,
                ).wait_recv()

        jo = lax.broadcasted_iota(jnp.int32, (M, 1), 0)
        in_keep = (jo >= off_keep) & (jo < off_keep + n_keep)
        out_ref[...] = jnp.where(in_keep, keep_rows, rbuf_ref[...])

    return pl.pallas_call(
        body,
        out_shape=jax.ShapeDtypeStruct((M, N), jnp.float32),
        in_specs=[
            pl.BlockSpec(memory_space=pltpu.VMEM),
            pl.BlockSpec(memory_space=pltpu.VMEM),
        ],
        out_specs=pl.BlockSpec(memory_space=pltpu.VMEM),
        scratch_shapes=[
            pltpu.VMEM((M, N), jnp.float32),
            pltpu.VMEM((M, N), jnp.float32),
            pltpu.SemaphoreType.DMA((NCH,)),
            pltpu.SemaphoreType.DMA((NCH,)),
        ],
        compiler_params=pltpu.CompilerParams(collective_id=0),
    )(x, dest2)
